# baseline (speedup 1.0000x reference)
import functools

import jax
import jax.numpy as jnp
from jax import lax
from jax.experimental import pallas as pl
from jax.experimental.pallas import tpu as pltpu
from jax.experimental.pallas import tpu_sc as plsc

N = 10000
D = 128
E = 320000
NC = 2
NS = 16
NW = NC * NS
L = 16

F32 = jnp.float32
NP = 10240


def _mesh():
    return plsc.VectorSubcoreMesh(
        core_axis_name="c", subcore_axis_name="s", num_cores=NC, num_subcores=NS
    )


def _tc_prep_body(xp_ref, xa_ref, wc_ref, asc_ref, adc_ref,
                  ww_ref, asw_ref, adw_ref, wwb_ref, aswb_ref, adwb_ref,
                  hc_ref, hw_ref, hwb_ref,
                  sc_ref, tc_ref, sw_ref, tw_ref, swb_ref, twb_ref):
    xp = xp_ref[...]
    xa = xa_ref[...]
    wc = wc_ref[...]
    ww = ww_ref[...]
    wwb = wwb_ref[...]
    hc = jnp.dot(xp, wc, preferred_element_type=F32)
    hw = jnp.dot(xa, ww, preferred_element_type=F32)
    hwb = jnp.dot(xp, wwb, preferred_element_type=F32)
    hc_ref[...] = hc
    hw_ref[...] = hw
    hwb_ref[...] = hwb

    def mv(h, a):
        return jnp.sum(h * a[None, :], axis=1)

    sc_ref[...] = mv(hc, asc_ref[...])
    tc_ref[...] = mv(hc, adc_ref[...])
    sw_ref[...] = mv(hw, asw_ref[...])
    tw_ref[...] = mv(xp, jnp.sum(ww * adw_ref[...][None, :], axis=1))
    swb_ref[...] = mv(hwb, aswb_ref[...])
    twb_ref[...] = mv(xa, jnp.sum(wwb * adwb_ref[...][None, :], axis=1))


def _tc_prep(xp, xa, wc, asc, adc, ww, asw, adw, wwb, aswb, adwb):
    mat = jax.ShapeDtypeStruct((N, D), F32)
    vec = jax.ShapeDtypeStruct((N,), F32)
    return pl.pallas_call(
        _tc_prep_body,
        out_shape=(mat, mat, mat, vec, vec, vec, vec, vec, vec),
    )(xp, xa, wc, asc, adc, ww, asw, adw, wwb, aswb, adwb)


C2 = 512
NG2 = C2 // 128
NCH2 = E // C2
BC2 = NCH2 // NW
REM2 = NCH2 - BC2 * NW


def _sc_pass1_body(sc_r, tc_r, sw_r, tw_r, swb_r, twb_r,
                   srcc, dstc, srcw, dstw, srcwb, dstwb,
                   exc, exw, exwb, dpc, dpw, dpwb,
                   s_tab, t_tab, src_b, dst_b, ex_b, zb, d0, d1, d2):
    cid = lax.axis_index("c")
    sid = lax.axis_index("s")
    wid = sid * NC + cid

    def zb_zero(i, carry):
        zb[pl.ds(i * L, L)] = jnp.zeros((L,), F32)
        return carry

    lax.fori_loop(0, 640 // L, zb_zero, 0)

    for dsh in (d0, d1, d2):
        pltpu.sync_copy(zb, dsh.at[pl.ds(sid * 640, 640)])

    plsc.subcore_barrier()

    rels = ((sc_r, tc_r, srcc, dstc, exc, d0),
            (sw_r, tw_r, srcw, dstw, exw, d1),
            (swb_r, twb_r, srcwb, dstwb, exwb, d2))
    for s_r, t_r, src_r, dst_r, ex_r, dsh in rels:
        pltpu.sync_copy(s_r, s_tab)
        pltpu.sync_copy(t_r, t_tab)

        def mx_body(i, cur):
            return jnp.maximum(cur, s_tab[pl.ds(i * L, L)])

        mv = lax.fori_loop(0, N // L, mx_body, jnp.full((L,), -1e30, F32))
        sm = mv[0]
        for i in range(1, L):
            sm = jnp.maximum(sm, mv[i])

        nch = BC2 + jnp.where(wid < REM2, 1, 0)

        def chunk(k, carry):
            base = (wid + k * NW) * C2
            pltpu.sync_copy(src_r.at[pl.ds(base, C2)], src_b)
            for g in range(NG2):
                pltpu.sync_copy(dst_r.at[pl.ds(base + g * 128, 128)], dst_b.at[g])
            for g in range(NG2):
                for j in range(128 // L):
                    sl = pl.ds(g * 128 + j * L, L)
                    iv_s = src_b[sl]
                    iv_d = dst_b[g, pl.ds(j * L, L)]
                    sv = plsc.load_gather(s_tab, [iv_s])
                    tv = plsc.load_gather(t_tab, [iv_d])
                    z = sv + tv
                    e = jnp.maximum(z, 0.2 * z)
                    mz = sm + tv
                    mm = jnp.maximum(mz, 0.2 * mz)
                    ex_b[sl] = jnp.exp(e - mm)
            pltpu.sync_copy(ex_b, ex_r.at[pl.ds(base, C2)])
            for g in range(NG2):
                pltpu.sync_copy(ex_b.at[pl.ds(g * 128, 128)],
                                dsh.at[dst_b.at[g]], add=True)
            return carry

        lax.fori_loop(0, nch, chunk, 0)

    plsc.subcore_barrier()

    for dsh, dp in ((d0, dpc), (d1, dpw), (d2, dpwb)):
        pltpu.sync_copy(dsh.at[pl.ds(sid * 640, 640)],
                        dp.at[cid, pl.ds(sid * 640, 640)])


def _sc_pass1(s_c, t_c, s_w, t_w, s_wb, t_wb,
              src_c, dst_c, src_w, dst_w, src_wb, dst_wb):
    ev = jax.ShapeDtypeStruct((E,), F32)
    dp = jax.ShapeDtypeStruct((NC, NP), F32)
    f = pl.kernel(
        _sc_pass1_body,
        out_type=(ev, ev, ev, dp, dp, dp),
        mesh=_mesh(),
        compiler_params=pltpu.CompilerParams(needs_layout_passes=False),
        scratch_types=[
            pltpu.VMEM((N,), F32),
            pltpu.VMEM((N,), F32),
            pltpu.VMEM((C2,), jnp.int32),
            pltpu.VMEM((NG2, 128), jnp.int32),
            pltpu.VMEM((C2,), F32),
            pltpu.VMEM((640,), F32),
            pltpu.VMEM_SHARED((NP,), F32),
            pltpu.VMEM_SHARED((NP,), F32),
            pltpu.VMEM_SHARED((NP,), F32),
        ],
    )
    return f(s_c, t_c, s_w, t_w, s_wb, t_wb,
             src_c, dst_c, src_w, dst_w, src_wb, dst_wb)


def _tc_densum_body(dc_ref, dw_ref, dwb_ref, oc_ref, ow_ref, owb_ref):
    oc_ref[...] = dc_ref[0] + dc_ref[1]
    ow_ref[...] = dw_ref[0] + dw_ref[1]
    owb_ref[...] = dwb_ref[0] + dwb_ref[1]


def _tc_densum(dp_c, dp_w, dp_wb):
    v = jax.ShapeDtypeStruct((NP,), F32)
    return pl.pallas_call(_tc_densum_body, out_shape=(v, v, v))(dp_c, dp_w, dp_wb)


G = 128
C4 = G
NCH4 = E // C4
UCH = NCH4 // NS
REM4 = NCH4 - UCH * NS
NBUF = 4
HR = NP // NC
AR = HR + 80
DUMP = HR


def _sc_pass2_body(nrel, *refs):
    ins = refs[:5 * nrel]
    outs = refs[5 * nrel:6 * nrel + 1]
    scr = refs[6 * nrel + 1:]
    (den_tab, src_b, dst_b, dstl_b, ex_b, al_b, rows, zb, acc) = scr[:9]
    gsems = scr[9:9 + NBUF]
    ssems = scr[9 + NBUF:9 + 2 * NBUF]
    cid = lax.axis_index("c")
    sid = lax.axis_index("s")

    def zrow(i, carry):
        for q in range(D // L):
            zb[i, pl.ds(q * L, L)] = jnp.zeros((L,), F32)
        return carry

    lax.fori_loop(0, 16, zrow, 0)

    def zc(i, carry):
        pltpu.sync_copy(zb, acc.at[pl.ds(sid * (HR // NS) + i * 16, 16)])
        return carry

    lax.fori_loop(0, HR // NS // 16, zc, 0)

    @pl.when(sid == 0)
    def _():
        def zd(i, carry):
            pltpu.sync_copy(zb, acc.at[pl.ds(HR + i * 16, 16)])
            return carry

        lax.fori_loop(0, (AR - HR) // 16, zd, 0)

    plsc.subcore_barrier()

    for r in range(nrel):
        h, den_r, exh, srch, dsth = ins[5 * r:5 * r + 5]
        alph = outs[r]
        pltpu.sync_copy(den_r, den_tab)

        def fire(k, b):
            base = (sid + k * NS) * C4
            pltpu.sync_copy(srch.at[pl.ds(base, G)], src_b.at[b])
            pltpu.async_copy(h.at[src_b.at[b]],
                             rows.at[b], gsems[b])
            pltpu.sync_copy(dsth.at[pl.ds(base, G)], dst_b.at[b])
            pltpu.sync_copy(exh.at[pl.ds(base, C4)], ex_b.at[b])

        def drain_scatter(b):
            pltpu.make_async_copy(rows.at[b], acc.at[dstl_b.at[b]],
                                  ssems[b]).wait()

        def finish(k, b, tail):
            base = (sid + k * NS) * C4
            for j in range(G // L):
                sl = pl.ds(j * L, L)
                dv = dst_b[b, sl]
                dg = plsc.load_gather(den_tab, [dv])
                al_b[b, sl] = ex_b[b, sl] / (dg + 1e-16)
                lv = dv - cid * HR
                ok = (lv >= 0) & (lv < HR)
                dstl_b[b, sl] = jnp.where(ok, lv, DUMP)

            @pl.when(cid == 0)
            def _():
                pltpu.sync_copy(al_b.at[b], alph.at[pl.ds(base, C4)])

            pltpu.make_async_copy(h.at[src_b.at[b]], rows.at[b],
                                  gsems[b]).wait()

            def sgrp(g16, carry2):
                al16 = al_b[b, pl.ds(g16 * L, L)]
                for jj in range(L):
                    a = al16[jj]
                    j = g16 * L + jj
                    for q in range(D // L):
                        sl2 = pl.ds(q * L, L)
                        rows[b, j, sl2] = rows[b, j, sl2] * a
                return carry2

            lax.fori_loop(0, C4 // L, sgrp, 0)
            pltpu.sync_copy(rows.at[b], acc.at[dstl_b.at[b]], add=True)

        fire(0, 0)
        fire(1, 1)

        def pipe(i, carry):
            for b in range(NBUF):
                m = 4 * i + b
                mf = m + 2
                bf = (b + 2) % NBUF

                @pl.when(mf < UCH)
                def _():
                    fire(mf, bf)

                finish(m, b, tail=False)
            return carry

        lax.fori_loop(0, UCH // NBUF, pipe, 0)

        @pl.when(sid < REM4)
        def _():
            fire(UCH, 0)
            finish(UCH, 0, tail=True)

    plsc.subcore_barrier()
    outp = outs[nrel]

    def fl(i, carry):
        off = sid * (HR // NS) + i * 80
        pltpu.sync_copy(acc.at[pl.ds(off, 80)],
                        outp.at[pl.ds(cid * HR + off, 80)])
        return carry

    lax.fori_loop(0, HR // NS // 80, fl, 0)


def _sc_pass2(nrel, args):
    ev = jax.ShapeDtypeStruct((E,), F32)
    op = jax.ShapeDtypeStruct((NP, D), F32)
    f = pl.kernel(
        functools.partial(_sc_pass2_body, nrel),
        out_type=tuple([ev] * nrel) + (op,),
        mesh=_mesh(),
        compiler_params=pltpu.CompilerParams(needs_layout_passes=False),
        scratch_types=[
            pltpu.VMEM((NP,), F32),
            pltpu.VMEM((NBUF, G), jnp.int32),
            pltpu.VMEM((NBUF, G), jnp.int32),
            pltpu.VMEM((NBUF, G), jnp.int32),
            pltpu.VMEM((NBUF, C4), F32),
            pltpu.VMEM((NBUF, C4), F32),
            pltpu.VMEM((NBUF, C4, D), F32),
            pltpu.VMEM((16, D), F32),
            pltpu.VMEM_SHARED((AR, D), F32),
        ] + [pltpu.SemaphoreType.DMA] * (2 * NBUF),
    )
    return f(*args)


def kernel(x_paper, x_author, ei_cites, ei_writes, ei_written_by,
           W_cites, a_src_cites, a_dst_cites,
           W_writes, a_src_writes, a_dst_writes,
           W_wb, a_src_wb, a_dst_wb):
    src_c, dst_c = ei_cites[0], ei_cites[1]
    src_w, dst_w = ei_writes[0], ei_writes[1]
    src_wb, dst_wb = ei_written_by[0], ei_written_by[1]

    (hc, hw, hwb, s_c, t_c, s_w, t_w, s_wb, t_wb) = _tc_prep(
        x_paper, x_author, W_cites, a_src_cites, a_dst_cites,
        W_writes, a_src_writes, a_dst_writes, W_wb, a_src_wb, a_dst_wb)

    ex_c, ex_w, ex_wb, dp_c, dp_w, dp_wb = _sc_pass1(
        s_c, t_c, s_w, t_w, s_wb, t_wb,
        src_c, dst_c, src_w, dst_w, src_wb, dst_wb)

    den_c, den_w, den_wb = _tc_densum(dp_c, dp_w, dp_wb)

    alpha_c, alpha_w, outp = _sc_pass2(
        2, (hc, den_c, ex_c, src_c, dst_c,
            hw, den_w, ex_w, src_w, dst_w))
    alpha_wb, outa = _sc_pass2(
        1, (hwb, den_wb, ex_wb, src_wb, dst_wb))

    return (outp[:N], outa[:N], alpha_c, alpha_w, alpha_wb)


# speedup vs baseline: 1.0056x; 1.0056x over previous
import functools

import jax
import jax.numpy as jnp
from jax import lax
from jax.experimental import pallas as pl
from jax.experimental.pallas import tpu as pltpu
from jax.experimental.pallas import tpu_sc as plsc

N = 10000
D = 128
E = 320000
NC = 2
NS = 16
NW = NC * NS
L = 16

F32 = jnp.float32
NP = 10240


def _mesh():
    return plsc.VectorSubcoreMesh(
        core_axis_name="c", subcore_axis_name="s", num_cores=NC, num_subcores=NS
    )


def _tc_prep_body(xp_ref, xa_ref, wc_ref, asc_ref, adc_ref,
                  ww_ref, asw_ref, adw_ref, wwb_ref, aswb_ref, adwb_ref,
                  hc_ref, hw_ref, hwb_ref,
                  sc_ref, tc_ref, sw_ref, tw_ref, swb_ref, twb_ref):
    xp = xp_ref[...]
    xa = xa_ref[...]
    wc = wc_ref[...]
    ww = ww_ref[...]
    wwb = wwb_ref[...]
    hc = jnp.dot(xp, wc, preferred_element_type=F32)
    hw = jnp.dot(xa, ww, preferred_element_type=F32)
    hwb = jnp.dot(xp, wwb, preferred_element_type=F32)
    hc_ref[...] = hc
    hw_ref[...] = hw
    hwb_ref[...] = hwb

    def mv(h, a):
        return jnp.sum(h * a[None, :], axis=1)

    sc_ref[...] = mv(hc, asc_ref[...])
    tc_ref[...] = mv(hc, adc_ref[...])
    sw_ref[...] = mv(hw, asw_ref[...])
    tw_ref[...] = mv(xp, jnp.sum(ww * adw_ref[...][None, :], axis=1))
    swb_ref[...] = mv(hwb, aswb_ref[...])
    twb_ref[...] = mv(xa, jnp.sum(wwb * adwb_ref[...][None, :], axis=1))


def _tc_prep(xp, xa, wc, asc, adc, ww, asw, adw, wwb, aswb, adwb):
    mat = jax.ShapeDtypeStruct((N, D), F32)
    vec = jax.ShapeDtypeStruct((N,), F32)
    return pl.pallas_call(
        _tc_prep_body,
        out_shape=(mat, mat, mat, vec, vec, vec, vec, vec, vec),
    )(xp, xa, wc, asc, adc, ww, asw, adw, wwb, aswb, adwb)


C2 = 512
NG2 = C2 // 128
NCH2 = E // C2
BC2 = NCH2 // NW
REM2 = NCH2 - BC2 * NW


def _sc_pass1_body(sc_r, tc_r, sw_r, tw_r, swb_r, twb_r,
                   srcc, dstc, srcw, dstw, srcwb, dstwb,
                   exc, exw, exwb, dpc, dpw, dpwb,
                   s_tab, t_tab, src_b, dst_b, ex_b, zb, d0, d1, d2):
    cid = lax.axis_index("c")
    sid = lax.axis_index("s")
    wid = sid * NC + cid

    def zb_zero(i, carry):
        zb[pl.ds(i * L, L)] = jnp.zeros((L,), F32)
        return carry

    lax.fori_loop(0, 640 // L, zb_zero, 0)

    for dsh in (d0, d1, d2):
        pltpu.sync_copy(zb, dsh.at[pl.ds(sid * 640, 640)])

    plsc.subcore_barrier()

    rels = ((sc_r, tc_r, srcc, dstc, exc, d0),
            (sw_r, tw_r, srcw, dstw, exw, d1),
            (swb_r, twb_r, srcwb, dstwb, exwb, d2))
    for s_r, t_r, src_r, dst_r, ex_r, dsh in rels:
        pltpu.sync_copy(s_r, s_tab)
        pltpu.sync_copy(t_r, t_tab)

        def mx_body(i, cur):
            return jnp.maximum(cur, s_tab[pl.ds(i * L, L)])

        mv = lax.fori_loop(0, N // L, mx_body, jnp.full((L,), -1e30, F32))
        sm = mv[0]
        for i in range(1, L):
            sm = jnp.maximum(sm, mv[i])

        nch = BC2 + jnp.where(wid < REM2, 1, 0)

        def chunk(k, carry):
            base = (wid + k * NW) * C2
            pltpu.sync_copy(src_r.at[pl.ds(base, C2)], src_b)
            for g in range(NG2):
                pltpu.sync_copy(dst_r.at[pl.ds(base + g * 128, 128)], dst_b.at[g])
            for g in range(NG2):
                for j in range(128 // L):
                    sl = pl.ds(g * 128 + j * L, L)
                    iv_s = src_b[sl]
                    iv_d = dst_b[g, pl.ds(j * L, L)]
                    sv = plsc.load_gather(s_tab, [iv_s])
                    tv = plsc.load_gather(t_tab, [iv_d])
                    z = sv + tv
                    e = jnp.maximum(z, 0.2 * z)
                    mz = sm + tv
                    mm = jnp.maximum(mz, 0.2 * mz)
                    ex_b[sl] = jnp.exp(e - mm)
            pltpu.sync_copy(ex_b, ex_r.at[pl.ds(base, C2)])
            for g in range(NG2):
                pltpu.sync_copy(ex_b.at[pl.ds(g * 128, 128)],
                                dsh.at[dst_b.at[g]], add=True)
            return carry

        lax.fori_loop(0, nch, chunk, 0)

    plsc.subcore_barrier()

    for dsh, dp in ((d0, dpc), (d1, dpw), (d2, dpwb)):
        pltpu.sync_copy(dsh.at[pl.ds(sid * 640, 640)],
                        dp.at[cid, pl.ds(sid * 640, 640)])


def _sc_pass1(s_c, t_c, s_w, t_w, s_wb, t_wb,
              src_c, dst_c, src_w, dst_w, src_wb, dst_wb):
    ev = jax.ShapeDtypeStruct((E,), F32)
    dp = jax.ShapeDtypeStruct((NC, NP), F32)
    f = pl.kernel(
        _sc_pass1_body,
        out_type=(ev, ev, ev, dp, dp, dp),
        mesh=_mesh(),
        compiler_params=pltpu.CompilerParams(needs_layout_passes=False),
        scratch_types=[
            pltpu.VMEM((N,), F32),
            pltpu.VMEM((N,), F32),
            pltpu.VMEM((C2,), jnp.int32),
            pltpu.VMEM((NG2, 128), jnp.int32),
            pltpu.VMEM((C2,), F32),
            pltpu.VMEM((640,), F32),
            pltpu.VMEM_SHARED((NP,), F32),
            pltpu.VMEM_SHARED((NP,), F32),
            pltpu.VMEM_SHARED((NP,), F32),
        ],
    )
    return f(s_c, t_c, s_w, t_w, s_wb, t_wb,
             src_c, dst_c, src_w, dst_w, src_wb, dst_wb)


def _tc_densum_body(dc_ref, dw_ref, dwb_ref, oc_ref, ow_ref, owb_ref):
    oc_ref[...] = dc_ref[0] + dc_ref[1]
    ow_ref[...] = dw_ref[0] + dw_ref[1]
    owb_ref[...] = dwb_ref[0] + dwb_ref[1]


def _tc_densum(dp_c, dp_w, dp_wb):
    v = jax.ShapeDtypeStruct((NP,), F32)
    return pl.pallas_call(_tc_densum_body, out_shape=(v, v, v))(dp_c, dp_w, dp_wb)


G = 128
C4 = G
NCH4 = E // C4
UCH = NCH4 // NS
REM4 = NCH4 - UCH * NS
NBUF = 4
HR = NP // NC
AR = HR + 80
DUMP = HR


def _sc_pass2_body(nrel, *refs):
    ins = refs[:5 * nrel]
    outs = refs[5 * nrel:6 * nrel + 1]
    scr = refs[6 * nrel + 1:]
    (den_tab, src_b, dst_b, dstl_b, ex_b, al_b, rows, zb, acc) = scr[:9]
    gsems = scr[9:9 + NBUF]
    ssems = scr[9 + NBUF:9 + 2 * NBUF]
    cid = lax.axis_index("c")
    sid = lax.axis_index("s")

    def zrow(i, carry):
        for q in range(D // L):
            zb[i, pl.ds(q * L, L)] = jnp.zeros((L,), F32)
        return carry

    lax.fori_loop(0, 16, zrow, 0)

    def zc(i, carry):
        pltpu.sync_copy(zb, acc.at[pl.ds(sid * (HR // NS) + i * 16, 16)])
        return carry

    lax.fori_loop(0, HR // NS // 16, zc, 0)

    @pl.when(sid == 0)
    def _():
        def zd(i, carry):
            pltpu.sync_copy(zb, acc.at[pl.ds(HR + i * 16, 16)])
            return carry

        lax.fori_loop(0, (AR - HR) // 16, zd, 0)

    plsc.subcore_barrier()

    for r in range(nrel):
        h, den_r, exh, srch, dsth = ins[5 * r:5 * r + 5]
        alph = outs[r]
        pltpu.sync_copy(den_r, den_tab)

        def fire(k, b):
            base = (sid + k * NS) * C4
            pltpu.sync_copy(srch.at[pl.ds(base, G)], src_b.at[b])
            pltpu.async_copy(h.at[src_b.at[b]],
                             rows.at[b], gsems[b])
            pltpu.sync_copy(dsth.at[pl.ds(base, G)], dst_b.at[b])
            pltpu.sync_copy(exh.at[pl.ds(base, C4)], ex_b.at[b])

        def drain_scatter(b):
            pltpu.make_async_copy(rows.at[b], acc.at[dstl_b.at[b]],
                                  ssems[b]).wait()

        def finish(k, b, tail):
            base = (sid + k * NS) * C4
            for j in range(G // L):
                sl = pl.ds(j * L, L)
                dv = dst_b[b, sl]
                dg = plsc.load_gather(den_tab, [dv])
                al_b[b, sl] = ex_b[b, sl] / (dg + 1e-16)
                lv = dv - cid * HR
                ok = (lv >= 0) & (lv < HR)
                dstl_b[b, sl] = jnp.where(ok, lv, DUMP + (dv & 63))

            @pl.when(cid == 0)
            def _():
                pltpu.sync_copy(al_b.at[b], alph.at[pl.ds(base, C4)])

            pltpu.make_async_copy(h.at[src_b.at[b]], rows.at[b],
                                  gsems[b]).wait()

            def sgrp(g16, carry2):
                al16 = al_b[b, pl.ds(g16 * L, L)]
                for jj in range(L):
                    a = al16[jj]
                    j = g16 * L + jj
                    for q in range(D // L):
                        sl2 = pl.ds(q * L, L)
                        rows[b, j, sl2] = rows[b, j, sl2] * a
                return carry2

            lax.fori_loop(0, C4 // L, sgrp, 0)
            pltpu.sync_copy(rows.at[b], acc.at[dstl_b.at[b]], add=True)

        fire(0, 0)
        fire(1, 1)

        def pipe(i, carry):
            for b in range(NBUF):
                m = 4 * i + b
                mf = m + 2
                bf = (b + 2) % NBUF

                @pl.when(mf < UCH)
                def _():
                    fire(mf, bf)

                finish(m, b, tail=False)
            return carry

        lax.fori_loop(0, UCH // NBUF, pipe, 0)

        @pl.when(sid < REM4)
        def _():
            fire(UCH, 0)
            finish(UCH, 0, tail=True)

    plsc.subcore_barrier()
    outp = outs[nrel]

    def fl(i, carry):
        off = sid * (HR // NS) + i * 80
        pltpu.sync_copy(acc.at[pl.ds(off, 80)],
                        outp.at[pl.ds(cid * HR + off, 80)])
        return carry

    lax.fori_loop(0, HR // NS // 80, fl, 0)


def _sc_pass2(nrel, args):
    ev = jax.ShapeDtypeStruct((E,), F32)
    op = jax.ShapeDtypeStruct((NP, D), F32)
    f = pl.kernel(
        functools.partial(_sc_pass2_body, nrel),
        out_type=tuple([ev] * nrel) + (op,),
        mesh=_mesh(),
        compiler_params=pltpu.CompilerParams(needs_layout_passes=False),
        scratch_types=[
            pltpu.VMEM((NP,), F32),
            pltpu.VMEM((NBUF, G), jnp.int32),
            pltpu.VMEM((NBUF, G), jnp.int32),
            pltpu.VMEM((NBUF, G), jnp.int32),
            pltpu.VMEM((NBUF, C4), F32),
            pltpu.VMEM((NBUF, C4), F32),
            pltpu.VMEM((NBUF, C4, D), F32),
            pltpu.VMEM((16, D), F32),
            pltpu.VMEM_SHARED((AR, D), F32),
        ] + [pltpu.SemaphoreType.DMA] * (2 * NBUF),
    )
    return f(*args)


def kernel(x_paper, x_author, ei_cites, ei_writes, ei_written_by,
           W_cites, a_src_cites, a_dst_cites,
           W_writes, a_src_writes, a_dst_writes,
           W_wb, a_src_wb, a_dst_wb):
    src_c, dst_c = ei_cites[0], ei_cites[1]
    src_w, dst_w = ei_writes[0], ei_writes[1]
    src_wb, dst_wb = ei_written_by[0], ei_written_by[1]

    (hc, hw, hwb, s_c, t_c, s_w, t_w, s_wb, t_wb) = _tc_prep(
        x_paper, x_author, W_cites, a_src_cites, a_dst_cites,
        W_writes, a_src_writes, a_dst_writes, W_wb, a_src_wb, a_dst_wb)

    ex_c, ex_w, ex_wb, dp_c, dp_w, dp_wb = _sc_pass1(
        s_c, t_c, s_w, t_w, s_wb, t_wb,
        src_c, dst_c, src_w, dst_w, src_wb, dst_wb)

    den_c, den_w, den_wb = _tc_densum(dp_c, dp_w, dp_wb)

    alpha_c, alpha_w, outp = _sc_pass2(
        2, (hc, den_c, ex_c, src_c, dst_c,
            hw, den_w, ex_w, src_w, dst_w))
    alpha_wb, outa = _sc_pass2(
        1, (hwb, den_wb, ex_wb, src_wb, dst_wb))

    return (outp[:N], outa[:N], alpha_c, alpha_w, alpha_wb)


# speedup vs baseline: 1.1954x; 1.1887x over previous
import functools

import jax
import jax.numpy as jnp
from jax import lax
from jax.experimental import pallas as pl
from jax.experimental.pallas import tpu as pltpu
from jax.experimental.pallas import tpu_sc as plsc

N = 10000
D = 128
E = 320000
NC = 2
NS = 16
NW = NC * NS
L = 16

F32 = jnp.float32
NP = 10240


def _mesh():
    return plsc.VectorSubcoreMesh(
        core_axis_name="c", subcore_axis_name="s", num_cores=NC, num_subcores=NS
    )


def _tc_prep_body(xp_ref, xa_ref, wc_ref, asc_ref, adc_ref,
                  ww_ref, asw_ref, adw_ref, wwb_ref, aswb_ref, adwb_ref,
                  hc_ref, hw_ref, hwb_ref,
                  sc_ref, tc_ref, sw_ref, tw_ref, swb_ref, twb_ref):
    xp = xp_ref[...]
    xa = xa_ref[...]
    wc = wc_ref[...]
    ww = ww_ref[...]
    wwb = wwb_ref[...]
    hc = jnp.dot(xp, wc, preferred_element_type=F32)
    hw = jnp.dot(xa, ww, preferred_element_type=F32)
    hwb = jnp.dot(xp, wwb, preferred_element_type=F32)
    hc_ref[...] = hc
    hw_ref[...] = hw
    hwb_ref[...] = hwb

    def mv(h, a):
        return jnp.sum(h * a[None, :], axis=1)

    sc_ref[...] = mv(hc, asc_ref[...])
    tc_ref[...] = mv(hc, adc_ref[...])
    sw_ref[...] = mv(hw, asw_ref[...])
    tw_ref[...] = mv(xp, jnp.sum(ww * adw_ref[...][None, :], axis=1))
    swb_ref[...] = mv(hwb, aswb_ref[...])
    twb_ref[...] = mv(xa, jnp.sum(wwb * adwb_ref[...][None, :], axis=1))


def _tc_prep(xp, xa, wc, asc, adc, ww, asw, adw, wwb, aswb, adwb):
    mat = jax.ShapeDtypeStruct((N, D), F32)
    vec = jax.ShapeDtypeStruct((N,), F32)
    return pl.pallas_call(
        _tc_prep_body,
        out_shape=(mat, mat, mat, vec, vec, vec, vec, vec, vec),
    )(xp, xa, wc, asc, adc, ww, asw, adw, wwb, aswb, adwb)


C2 = 512
NG2 = C2 // 128
NCH2 = E // C2
BC2 = NCH2 // NW
REM2 = NCH2 - BC2 * NW


def _sc_pass1_body(sc_r, tc_r, sw_r, tw_r, swb_r, twb_r,
                   srcc, dstc, srcw, dstw, srcwb, dstwb,
                   exc, exw, exwb, dpc, dpw, dpwb,
                   s_tab, t_tab, src_b, dst_b, ex_b, zb, d0, d1, d2):
    cid = lax.axis_index("c")
    sid = lax.axis_index("s")
    wid = sid * NC + cid

    def zb_zero(i, carry):
        zb[pl.ds(i * L, L)] = jnp.zeros((L,), F32)
        return carry

    lax.fori_loop(0, 640 // L, zb_zero, 0)

    for dsh in (d0, d1, d2):
        pltpu.sync_copy(zb, dsh.at[pl.ds(sid * 640, 640)])

    plsc.subcore_barrier()

    rels = ((sc_r, tc_r, srcc, dstc, exc, d0),
            (sw_r, tw_r, srcw, dstw, exw, d1),
            (swb_r, twb_r, srcwb, dstwb, exwb, d2))
    for s_r, t_r, src_r, dst_r, ex_r, dsh in rels:
        pltpu.sync_copy(s_r, s_tab)
        pltpu.sync_copy(t_r, t_tab)

        def mx_body(i, cur):
            return jnp.maximum(cur, s_tab[pl.ds(i * L, L)])

        mv = lax.fori_loop(0, N // L, mx_body, jnp.full((L,), -1e30, F32))
        sm = mv[0]
        for i in range(1, L):
            sm = jnp.maximum(sm, mv[i])

        nch = BC2 + jnp.where(wid < REM2, 1, 0)

        def chunk(k, carry):
            base = (wid + k * NW) * C2
            pltpu.sync_copy(src_r.at[pl.ds(base, C2)], src_b)
            for g in range(NG2):
                pltpu.sync_copy(dst_r.at[pl.ds(base + g * 128, 128)], dst_b.at[g])
            for g in range(NG2):
                for j in range(128 // L):
                    sl = pl.ds(g * 128 + j * L, L)
                    iv_s = src_b[sl]
                    iv_d = dst_b[g, pl.ds(j * L, L)]
                    sv = plsc.load_gather(s_tab, [iv_s])
                    tv = plsc.load_gather(t_tab, [iv_d])
                    z = sv + tv
                    e = jnp.maximum(z, 0.2 * z)
                    mz = sm + tv
                    mm = jnp.maximum(mz, 0.2 * mz)
                    ex_b[sl] = jnp.exp(e - mm)
            pltpu.sync_copy(ex_b, ex_r.at[pl.ds(base, C2)])
            for g in range(NG2):
                pltpu.sync_copy(ex_b.at[pl.ds(g * 128, 128)],
                                dsh.at[dst_b.at[g]], add=True)
            return carry

        lax.fori_loop(0, nch, chunk, 0)

    plsc.subcore_barrier()

    for dsh, dp in ((d0, dpc), (d1, dpw), (d2, dpwb)):
        pltpu.sync_copy(dsh.at[pl.ds(sid * 640, 640)],
                        dp.at[cid, pl.ds(sid * 640, 640)])


def _sc_pass1(s_c, t_c, s_w, t_w, s_wb, t_wb,
              src_c, dst_c, src_w, dst_w, src_wb, dst_wb):
    ev = jax.ShapeDtypeStruct((E,), F32)
    dp = jax.ShapeDtypeStruct((NC, NP), F32)
    f = pl.kernel(
        _sc_pass1_body,
        out_type=(ev, ev, ev, dp, dp, dp),
        mesh=_mesh(),
        compiler_params=pltpu.CompilerParams(needs_layout_passes=False),
        scratch_types=[
            pltpu.VMEM((N,), F32),
            pltpu.VMEM((N,), F32),
            pltpu.VMEM((C2,), jnp.int32),
            pltpu.VMEM((NG2, 128), jnp.int32),
            pltpu.VMEM((C2,), F32),
            pltpu.VMEM((640,), F32),
            pltpu.VMEM_SHARED((NP,), F32),
            pltpu.VMEM_SHARED((NP,), F32),
            pltpu.VMEM_SHARED((NP,), F32),
        ],
    )
    return f(s_c, t_c, s_w, t_w, s_wb, t_wb,
             src_c, dst_c, src_w, dst_w, src_wb, dst_wb)


def _tc_densum_body(dc_ref, dw_ref, dwb_ref, oc_ref, ow_ref, owb_ref):
    oc_ref[...] = dc_ref[0] + dc_ref[1]
    ow_ref[...] = dw_ref[0] + dw_ref[1]
    owb_ref[...] = dwb_ref[0] + dwb_ref[1]


def _tc_densum(dp_c, dp_w, dp_wb):
    v = jax.ShapeDtypeStruct((NP,), F32)
    return pl.pallas_call(_tc_densum_body, out_shape=(v, v, v))(dp_c, dp_w, dp_wb)


G = 128
C4 = G
NCH4 = E // C4
UCH = NCH4 // NS
REM4 = NCH4 - UCH * NS
NBUF = 4
HR = NP // NC
AR = HR + 80
DUMP = HR


def _sc_pass2_body(nrel, *refs):
    ins = refs[:5 * nrel]
    outs = refs[5 * nrel:6 * nrel + 1]
    scr = refs[6 * nrel + 1:]
    (den_tab, src_b, dst_b, dstl_b, ex_b, al_b, rows, zb, acc) = scr[:9]
    gsems = scr[9:9 + NBUF]
    ssems = scr[9 + NBUF:9 + 2 * NBUF]
    cid = lax.axis_index("c")
    sid = lax.axis_index("s")

    def zrow(i, carry):
        for q in range(D // L):
            zb[i, pl.ds(q * L, L)] = jnp.zeros((L,), F32)
        return carry

    lax.fori_loop(0, 16, zrow, 0)

    def zc(i, carry):
        pltpu.sync_copy(zb, acc.at[pl.ds(sid * (HR // NS) + i * 16, 16)])
        return carry

    lax.fori_loop(0, HR // NS // 16, zc, 0)

    @pl.when(sid == 0)
    def _():
        def zd(i, carry):
            pltpu.sync_copy(zb, acc.at[pl.ds(HR + i * 16, 16)])
            return carry

        lax.fori_loop(0, (AR - HR) // 16, zd, 0)

    plsc.subcore_barrier()

    for r in range(nrel):
        h, den_r, exh, srch, dsth = ins[5 * r:5 * r + 5]
        alph = outs[r]
        pltpu.sync_copy(den_r, den_tab)

        def fire(k, b):
            base = (sid + k * NS) * C4
            pltpu.sync_copy(srch.at[pl.ds(base, G)], src_b.at[b])
            pltpu.async_copy(h.at[src_b.at[b]],
                             rows.at[b], gsems[b])
            pltpu.sync_copy(dsth.at[pl.ds(base, G)], dst_b.at[b])
            pltpu.sync_copy(exh.at[pl.ds(base, C4)], ex_b.at[b])

        def drain_scatter(b):
            pltpu.make_async_copy(rows.at[b], acc.at[dstl_b.at[b]],
                                  ssems[b]).wait()

        def finish(k, b, tail):
            base = (sid + k * NS) * C4
            for j in range(G // L):
                sl = pl.ds(j * L, L)
                dv = dst_b[b, sl]
                dg = plsc.load_gather(den_tab, [dv])
                al_b[b, sl] = ex_b[b, sl] / (dg + 1e-16)
                lv = dv - cid * HR
                ok = (lv >= 0) & (lv < HR)
                dstl_b[b, sl] = jnp.where(ok, lv, DUMP + (dv & 63))

            @pl.when(cid == 0)
            def _():
                pltpu.sync_copy(al_b.at[b], alph.at[pl.ds(base, C4)])

            pltpu.make_async_copy(h.at[src_b.at[b]], rows.at[b],
                                  gsems[b]).wait()

            def sgrp(g16, carry2):
                al16 = al_b[b, pl.ds(g16 * L, L)]
                for jj in range(L):
                    a = al16[jj]
                    j = g16 * L + jj
                    for q in range(D // L):
                        sl2 = pl.ds(q * L, L)
                        rows[b, j, sl2] = rows[b, j, sl2] * a
                return carry2

            lax.fori_loop(0, C4 // L, sgrp, 0)
            if tail:
                pltpu.sync_copy(rows.at[b], acc.at[dstl_b.at[b]], add=True)
            else:
                pltpu.async_copy(rows.at[b], acc.at[dstl_b.at[b]],
                                 ssems[b], add=True)

        fire(0, 0)
        fire(1, 1)

        def pipe(i, carry):
            for b in range(NBUF):
                m = 4 * i + b
                mf = m + 2
                bf = (b + 2) % NBUF

                @pl.when(mf < UCH)
                def _():
                    @pl.when(mf >= NBUF)
                    def _():
                        drain_scatter(bf)

                    fire(mf, bf)

                finish(m, b, tail=False)
            return carry

        lax.fori_loop(0, UCH // NBUF, pipe, 0)

        for b in range(NBUF):
            drain_scatter(b)

        @pl.when(sid < REM4)
        def _():
            fire(UCH, 0)
            finish(UCH, 0, tail=True)

    plsc.subcore_barrier()
    outp = outs[nrel]

    def fl(i, carry):
        off = sid * (HR // NS) + i * 80
        pltpu.sync_copy(acc.at[pl.ds(off, 80)],
                        outp.at[pl.ds(cid * HR + off, 80)])
        return carry

    lax.fori_loop(0, HR // NS // 80, fl, 0)


def _sc_pass2(nrel, args):
    ev = jax.ShapeDtypeStruct((E,), F32)
    op = jax.ShapeDtypeStruct((NP, D), F32)
    f = pl.kernel(
        functools.partial(_sc_pass2_body, nrel),
        out_type=tuple([ev] * nrel) + (op,),
        mesh=_mesh(),
        compiler_params=pltpu.CompilerParams(needs_layout_passes=False),
        scratch_types=[
            pltpu.VMEM((NP,), F32),
            pltpu.VMEM((NBUF, G), jnp.int32),
            pltpu.VMEM((NBUF, G), jnp.int32),
            pltpu.VMEM((NBUF, G), jnp.int32),
            pltpu.VMEM((NBUF, C4), F32),
            pltpu.VMEM((NBUF, C4), F32),
            pltpu.VMEM((NBUF, C4, D), F32),
            pltpu.VMEM((16, D), F32),
            pltpu.VMEM_SHARED((AR, D), F32),
        ] + [pltpu.SemaphoreType.DMA] * (2 * NBUF),
    )
    return f(*args)


def kernel(x_paper, x_author, ei_cites, ei_writes, ei_written_by,
           W_cites, a_src_cites, a_dst_cites,
           W_writes, a_src_writes, a_dst_writes,
           W_wb, a_src_wb, a_dst_wb):
    src_c, dst_c = ei_cites[0], ei_cites[1]
    src_w, dst_w = ei_writes[0], ei_writes[1]
    src_wb, dst_wb = ei_written_by[0], ei_written_by[1]

    (hc, hw, hwb, s_c, t_c, s_w, t_w, s_wb, t_wb) = _tc_prep(
        x_paper, x_author, W_cites, a_src_cites, a_dst_cites,
        W_writes, a_src_writes, a_dst_writes, W_wb, a_src_wb, a_dst_wb)

    ex_c, ex_w, ex_wb, dp_c, dp_w, dp_wb = _sc_pass1(
        s_c, t_c, s_w, t_w, s_wb, t_wb,
        src_c, dst_c, src_w, dst_w, src_wb, dst_wb)

    den_c, den_w, den_wb = _tc_densum(dp_c, dp_w, dp_wb)

    alpha_c, alpha_w, outp = _sc_pass2(
        2, (hc, den_c, ex_c, src_c, dst_c,
            hw, den_w, ex_w, src_w, dst_w))
    alpha_wb, outa = _sc_pass2(
        1, (hwb, den_wb, ex_wb, src_wb, dst_wb))

    return (outp[:N], outa[:N], alpha_c, alpha_w, alpha_wb)


# speedup vs baseline: 1.6881x; 1.4122x over previous
import functools

import jax
import jax.numpy as jnp
from jax import lax
from jax.experimental import pallas as pl
from jax.experimental.pallas import tpu as pltpu
from jax.experimental.pallas import tpu_sc as plsc

N = 10000
D = 128
E = 320000
NC = 2
NS = 16
NW = NC * NS
L = 16

F32 = jnp.float32
NP = 10240


def _mesh():
    return plsc.VectorSubcoreMesh(
        core_axis_name="c", subcore_axis_name="s", num_cores=NC, num_subcores=NS
    )


def _tc_prep_body(xp_ref, xa_ref, wc_ref, asc_ref, adc_ref,
                  ww_ref, asw_ref, adw_ref, wwb_ref, aswb_ref, adwb_ref,
                  hc_ref, hw_ref, hwb_ref,
                  sc_ref, tc_ref, sw_ref, tw_ref, swb_ref, twb_ref):
    xp = xp_ref[...]
    xa = xa_ref[...]
    wc = wc_ref[...]
    ww = ww_ref[...]
    wwb = wwb_ref[...]
    hc = jnp.dot(xp, wc, preferred_element_type=F32)
    hw = jnp.dot(xa, ww, preferred_element_type=F32)
    hwb = jnp.dot(xp, wwb, preferred_element_type=F32)
    hc_ref[...] = hc
    hw_ref[...] = hw
    hwb_ref[...] = hwb

    def mv(h, a):
        return jnp.sum(h * a[None, :], axis=1)

    sc_ref[...] = mv(hc, asc_ref[...])
    tc_ref[...] = mv(hc, adc_ref[...])
    sw_ref[...] = mv(hw, asw_ref[...])
    tw_ref[...] = mv(xp, jnp.sum(ww * adw_ref[...][None, :], axis=1))
    swb_ref[...] = mv(hwb, aswb_ref[...])
    twb_ref[...] = mv(xa, jnp.sum(wwb * adwb_ref[...][None, :], axis=1))


def _tc_prep(xp, xa, wc, asc, adc, ww, asw, adw, wwb, aswb, adwb):
    mat = jax.ShapeDtypeStruct((N, D), F32)
    vec = jax.ShapeDtypeStruct((N,), F32)
    return pl.pallas_call(
        _tc_prep_body,
        out_shape=(mat, mat, mat, vec, vec, vec, vec, vec, vec),
    )(xp, xa, wc, asc, adc, ww, asw, adw, wwb, aswb, adwb)


C2 = 512
NG2 = C2 // 128
NCH2 = E // C2
BC2 = NCH2 // NW
REM2 = NCH2 - BC2 * NW


def _sc_pass1_body(sc_r, tc_r, sw_r, tw_r, swb_r, twb_r,
                   srcc, dstc, srcw, dstw, srcwb, dstwb,
                   exc, exw, exwb, dpc, dpw, dpwb,
                   s_tab, t_tab, src_b, dst_b, ex_b, zb, d0, d1, d2):
    cid = lax.axis_index("c")
    sid = lax.axis_index("s")
    wid = sid * NC + cid

    def zb_zero(i, carry):
        zb[pl.ds(i * L, L)] = jnp.zeros((L,), F32)
        return carry

    lax.fori_loop(0, 640 // L, zb_zero, 0)

    for dsh in (d0, d1, d2):
        pltpu.sync_copy(zb, dsh.at[pl.ds(sid * 640, 640)])

    plsc.subcore_barrier()

    rels = ((sc_r, tc_r, srcc, dstc, exc, d0),
            (sw_r, tw_r, srcw, dstw, exw, d1),
            (swb_r, twb_r, srcwb, dstwb, exwb, d2))
    for s_r, t_r, src_r, dst_r, ex_r, dsh in rels:
        pltpu.sync_copy(s_r, s_tab)
        pltpu.sync_copy(t_r, t_tab)

        def mx_body(i, cur):
            return jnp.maximum(cur, s_tab[pl.ds(i * L, L)])

        mv = lax.fori_loop(0, N // L, mx_body, jnp.full((L,), -1e30, F32))
        sm = mv[0]
        for i in range(1, L):
            sm = jnp.maximum(sm, mv[i])

        nch = BC2 + jnp.where(wid < REM2, 1, 0)

        def chunk(k, carry):
            base = (wid + k * NW) * C2
            pltpu.sync_copy(src_r.at[pl.ds(base, C2)], src_b)
            for g in range(NG2):
                pltpu.sync_copy(dst_r.at[pl.ds(base + g * 128, 128)], dst_b.at[g])
            for g in range(NG2):
                for j in range(128 // L):
                    sl = pl.ds(g * 128 + j * L, L)
                    iv_s = src_b[sl]
                    iv_d = dst_b[g, pl.ds(j * L, L)]
                    sv = plsc.load_gather(s_tab, [iv_s])
                    tv = plsc.load_gather(t_tab, [iv_d])
                    z = sv + tv
                    e = jnp.maximum(z, 0.2 * z)
                    mz = sm + tv
                    mm = jnp.maximum(mz, 0.2 * mz)
                    ex_b[sl] = jnp.exp(e - mm)
            pltpu.sync_copy(ex_b, ex_r.at[pl.ds(base, C2)])
            for g in range(NG2):
                pltpu.sync_copy(ex_b.at[pl.ds(g * 128, 128)],
                                dsh.at[dst_b.at[g]], add=True)
            return carry

        lax.fori_loop(0, nch, chunk, 0)

    plsc.subcore_barrier()

    for dsh, dp in ((d0, dpc), (d1, dpw), (d2, dpwb)):
        pltpu.sync_copy(dsh.at[pl.ds(sid * 640, 640)],
                        dp.at[cid, pl.ds(sid * 640, 640)])


def _sc_pass1(s_c, t_c, s_w, t_w, s_wb, t_wb,
              src_c, dst_c, src_w, dst_w, src_wb, dst_wb):
    ev = jax.ShapeDtypeStruct((E,), F32)
    dp = jax.ShapeDtypeStruct((NC, NP), F32)
    f = pl.kernel(
        _sc_pass1_body,
        out_type=(ev, ev, ev, dp, dp, dp),
        mesh=_mesh(),
        compiler_params=pltpu.CompilerParams(needs_layout_passes=False),
        scratch_types=[
            pltpu.VMEM((N,), F32),
            pltpu.VMEM((N,), F32),
            pltpu.VMEM((C2,), jnp.int32),
            pltpu.VMEM((NG2, 128), jnp.int32),
            pltpu.VMEM((C2,), F32),
            pltpu.VMEM((640,), F32),
            pltpu.VMEM_SHARED((NP,), F32),
            pltpu.VMEM_SHARED((NP,), F32),
            pltpu.VMEM_SHARED((NP,), F32),
        ],
    )
    return f(s_c, t_c, s_w, t_w, s_wb, t_wb,
             src_c, dst_c, src_w, dst_w, src_wb, dst_wb)


def _tc_densum_body(dc_ref, dw_ref, dwb_ref, oc_ref, ow_ref, owb_ref):
    oc_ref[...] = dc_ref[0] + dc_ref[1]
    ow_ref[...] = dw_ref[0] + dw_ref[1]
    owb_ref[...] = dwb_ref[0] + dwb_ref[1]


def _tc_densum(dp_c, dp_w, dp_wb):
    v = jax.ShapeDtypeStruct((NP,), F32)
    return pl.pallas_call(_tc_densum_body, out_shape=(v, v, v))(dp_c, dp_w, dp_wb)


G = 128
C4 = G
NCH4 = E // C4
UCH = NCH4 // NS
REM4 = NCH4 - UCH * NS
NBUF = 4
HR = NP // NC
AR = HR + 80
DUMP = HR


def _sc_pass2_body(nrel, *refs):
    ins = refs[:5 * nrel]
    outs = refs[5 * nrel:6 * nrel + 1]
    scr = refs[6 * nrel + 1:]
    (den_tab, src_b, dst_b, dstl_b, ex_b, al_b, rows, zb, acc) = scr[:9]
    gsems = scr[9:9 + NBUF]
    ssems = scr[9 + NBUF:9 + 2 * NBUF]
    isems = scr[9 + 2 * NBUF:9 + 3 * NBUF]
    cid = lax.axis_index("c")
    sid = lax.axis_index("s")

    def zrow(i, carry):
        for q in range(D // L):
            zb[i, pl.ds(q * L, L)] = jnp.zeros((L,), F32)
        return carry

    lax.fori_loop(0, 16, zrow, 0)

    def zc(i, carry):
        pltpu.sync_copy(zb, acc.at[pl.ds(sid * (HR // NS) + i * 16, 16)])
        return carry

    lax.fori_loop(0, HR // NS // 16, zc, 0)

    @pl.when(sid == 0)
    def _():
        def zd(i, carry):
            pltpu.sync_copy(zb, acc.at[pl.ds(HR + i * 16, 16)])
            return carry

        lax.fori_loop(0, (AR - HR) // 16, zd, 0)

    plsc.subcore_barrier()

    for r in range(nrel):
        h, den_r, exh, srch, dsth = ins[5 * r:5 * r + 5]
        alph = outs[r]
        pltpu.sync_copy(den_r, den_tab)

        def fire(k, b):
            base = (sid + k * NS) * C4
            pltpu.sync_copy(srch.at[pl.ds(base, G)], src_b.at[b])
            pltpu.async_copy(h.at[src_b.at[b]],
                             rows.at[b], gsems[b])
            pltpu.async_copy(dsth.at[pl.ds(base, G)], dst_b.at[b], isems[b])
            pltpu.async_copy(exh.at[pl.ds(base, C4)], ex_b.at[b], isems[b])

        def drain_scatter(b):
            pltpu.make_async_copy(rows.at[b], acc.at[dstl_b.at[b]],
                                  ssems[b]).wait()

        def finish(k, b, tail):
            base = (sid + k * NS) * C4
            pltpu.make_async_copy(dsth.at[pl.ds(base, G)], dst_b.at[b],
                                  isems[b]).wait()
            pltpu.make_async_copy(exh.at[pl.ds(base, C4)], ex_b.at[b],
                                  isems[b]).wait()
            for j in range(G // L):
                sl = pl.ds(j * L, L)
                dv = dst_b[b, sl]
                dg = plsc.load_gather(den_tab, [dv])
                al_b[b, sl] = ex_b[b, sl] / (dg + 1e-16)
                lv = dv - cid * HR
                ok = (lv >= 0) & (lv < HR)
                dstl_b[b, sl] = jnp.where(ok, lv, DUMP + (dv & 63))

            @pl.when(cid == 0)
            def _():
                pltpu.sync_copy(al_b.at[b], alph.at[pl.ds(base, C4)])

            pltpu.make_async_copy(h.at[src_b.at[b]], rows.at[b],
                                  gsems[b]).wait()

            def sgrp(g16, carry2):
                al16 = al_b[b, pl.ds(g16 * L, L)]
                for jj in range(L):
                    a = al16[jj]
                    j = g16 * L + jj
                    for q in range(D // L):
                        sl2 = pl.ds(q * L, L)
                        rows[b, j, sl2] = rows[b, j, sl2] * a
                return carry2

            lax.fori_loop(0, C4 // L, sgrp, 0)
            if tail:
                pltpu.sync_copy(rows.at[b], acc.at[dstl_b.at[b]], add=True)
            else:
                pltpu.async_copy(rows.at[b], acc.at[dstl_b.at[b]],
                                 ssems[b], add=True)

        fire(0, 0)
        fire(1, 1)

        def pipe(i, carry):
            for b in range(NBUF):
                m = 4 * i + b
                mf = m + 2
                bf = (b + 2) % NBUF

                @pl.when(mf < UCH)
                def _():
                    @pl.when(mf >= NBUF)
                    def _():
                        drain_scatter(bf)

                    fire(mf, bf)

                finish(m, b, tail=False)
            return carry

        lax.fori_loop(0, UCH // NBUF, pipe, 0)

        for b in range(NBUF):
            drain_scatter(b)

        @pl.when(sid < REM4)
        def _():
            fire(UCH, 0)
            finish(UCH, 0, tail=True)

    plsc.subcore_barrier()
    outp = outs[nrel]

    def fl(i, carry):
        off = sid * (HR // NS) + i * 80
        pltpu.sync_copy(acc.at[pl.ds(off, 80)],
                        outp.at[pl.ds(cid * HR + off, 80)])
        return carry

    lax.fori_loop(0, HR // NS // 80, fl, 0)


def _sc_pass2(nrel, args):
    ev = jax.ShapeDtypeStruct((E,), F32)
    op = jax.ShapeDtypeStruct((NP, D), F32)
    f = pl.kernel(
        functools.partial(_sc_pass2_body, nrel),
        out_type=tuple([ev] * nrel) + (op,),
        mesh=_mesh(),
        compiler_params=pltpu.CompilerParams(needs_layout_passes=False),
        scratch_types=[
            pltpu.VMEM((NP,), F32),
            pltpu.VMEM((NBUF, G), jnp.int32),
            pltpu.VMEM((NBUF, G), jnp.int32),
            pltpu.VMEM((NBUF, G), jnp.int32),
            pltpu.VMEM((NBUF, C4), F32),
            pltpu.VMEM((NBUF, C4), F32),
            pltpu.VMEM((NBUF, C4, D), F32),
            pltpu.VMEM((16, D), F32),
            pltpu.VMEM_SHARED((AR, D), F32),
        ] + [pltpu.SemaphoreType.DMA] * (3 * NBUF),
    )
    return f(*args)


def kernel(x_paper, x_author, ei_cites, ei_writes, ei_written_by,
           W_cites, a_src_cites, a_dst_cites,
           W_writes, a_src_writes, a_dst_writes,
           W_wb, a_src_wb, a_dst_wb):
    src_c, dst_c = ei_cites[0], ei_cites[1]
    src_w, dst_w = ei_writes[0], ei_writes[1]
    src_wb, dst_wb = ei_written_by[0], ei_written_by[1]

    (hc, hw, hwb, s_c, t_c, s_w, t_w, s_wb, t_wb) = _tc_prep(
        x_paper, x_author, W_cites, a_src_cites, a_dst_cites,
        W_writes, a_src_writes, a_dst_writes, W_wb, a_src_wb, a_dst_wb)

    ex_c, ex_w, ex_wb, dp_c, dp_w, dp_wb = _sc_pass1(
        s_c, t_c, s_w, t_w, s_wb, t_wb,
        src_c, dst_c, src_w, dst_w, src_wb, dst_wb)

    den_c, den_w, den_wb = _tc_densum(dp_c, dp_w, dp_wb)

    alpha_c, alpha_w, outp = _sc_pass2(
        2, (hc, den_c, ex_c, src_c, dst_c,
            hw, den_w, ex_w, src_w, dst_w))
    alpha_wb, outa = _sc_pass2(
        1, (hwb, den_wb, ex_wb, src_wb, dst_wb))

    return (outp[:N], outa[:N], alpha_c, alpha_w, alpha_wb)
